# baseline (speedup 1.0000x reference)
import dataclasses
import functools

import jax
import jax.numpy as jnp
from jax import lax
from jax.experimental import pallas as pl
from jax.experimental.pallas import tpu as pltpu
from jax.experimental.pallas import tpu_sc as plsc

ZMAX = 100
NUM_FEATURES = 128
N_CONFIG = 20
N_ATOMS = 16384

_NC = 2
_NS = 16
_NW = _NC * _NS
_B_PER_W = N_ATOMS // _NW
_LANES = 16
_NK = NUM_FEATURES // _LANES

_CHUNK = 128
_N_CHUNK = _B_PER_W // _CHUNK

_ROWS_EACH = 8
_N_FULL_BUILDERS = ZMAX // _ROWS_EACH
_ROWS_LEFT = ZMAX - _ROWS_EACH * _N_FULL_BUILDERS

_sc_mesh = plsc.VectorSubcoreMesh(core_axis_name="c", subcore_axis_name="s")

_sc_params = pltpu.CompilerParams()
if "needs_layout_passes" in pltpu.CompilerParams.__dataclass_fields__:
    _sc_params = dataclasses.replace(_sc_params, needs_layout_passes=False)


def _full16(v):
    return jnp.full((_LANES,), v, jnp.int32)


def _build_rows(nrows, row0, ee_v, ec_v, cwt_v, tbuf):
    for j in range(nrows):
        accs = [ee_v[j, pl.ds(k * _LANES, _LANES)] for k in range(_NK)]
        e_lo = ec_v[j, pl.ds(0, _LANES)]
        e_hi = ec_v[j, pl.ds(N_CONFIG - _LANES, _LANES)]
        for cc in range(N_CONFIG):
            s = e_lo[cc] if cc < _LANES else e_hi[cc - (N_CONFIG - _LANES)]
            for k in range(_NK):
                accs[k] = accs[k] + s * cwt_v[cc, pl.ds(k * _LANES, _LANES)]
        for k in range(_NK):
            tbuf[j, pl.ds(k * _LANES, _LANES)] = accs[k]


@functools.partial(
    pl.kernel,
    mesh=_sc_mesh,
    compiler_params=_sc_params,
    out_type=jax.ShapeDtypeStruct((N_ATOMS, NUM_FEATURES), jnp.float32),
    scratch_types=[
        pltpu.VMEM_SHARED((ZMAX, NUM_FEATURES), jnp.float32),
        pltpu.VMEM((_N_CHUNK, _CHUNK, NUM_FEATURES), jnp.float32),
        pltpu.VMEM((_B_PER_W,), jnp.int32),
        pltpu.VMEM((NUM_FEATURES * N_CONFIG,), jnp.float32),
        pltpu.VMEM((N_CONFIG, NUM_FEATURES), jnp.float32),
        pltpu.VMEM((_ROWS_EACH, NUM_FEATURES), jnp.float32),
        pltpu.VMEM((_ROWS_EACH, N_CONFIG), jnp.float32),
        pltpu.VMEM((_ROWS_EACH, NUM_FEATURES), jnp.float32),
        pltpu.SemaphoreType.DMA((_N_CHUNK,)),
        pltpu.SemaphoreType.DMA((_N_CHUNK,)),
    ],
)
def _sc_lookup(idx_hbm, ee_hbm, ec_hbm, cw_hbm, out_hbm,
               table_sh, rows_v, idx_v, cw_v, cwt_v, ee_v, ec_v, tbuf,
               ssem, gsem):
    sid = lax.axis_index("s")
    wid = sid * _NC + lax.axis_index("c")
    base = wid * _B_PER_W
    pltpu.sync_copy(idx_hbm.at[pl.ds(base, _B_PER_W)], idx_v)

    pltpu.sync_copy(cw_hbm, cw_v)
    lane = jnp.arange(_LANES, dtype=jnp.int32)
    for cc in range(N_CONFIG):
        for k in range(_NK):
            v = plsc.load_gather(
                cw_v, [(lane + k * _LANES) * N_CONFIG + cc])
            cwt_v[cc, pl.ds(k * _LANES, _LANES)] = v

    @pl.when(sid < _N_FULL_BUILDERS)
    def _():
        row0 = sid * _ROWS_EACH
        pltpu.sync_copy(ee_hbm.at[pl.ds(row0, _ROWS_EACH)], ee_v)
        pltpu.sync_copy(ec_hbm.at[pl.ds(row0, _ROWS_EACH)], ec_v)
        _build_rows(_ROWS_EACH, row0, ee_v, ec_v, cwt_v, tbuf)
        pltpu.sync_copy(tbuf, table_sh.at[pl.ds(row0, _ROWS_EACH)])

    @pl.when(sid == _N_FULL_BUILDERS)
    def _():
        xrow = _ROWS_EACH * _N_FULL_BUILDERS
        pltpu.sync_copy(ee_hbm.at[pl.ds(xrow, _ROWS_LEFT)],
                        ee_v.at[pl.ds(0, _ROWS_LEFT)])
        pltpu.sync_copy(ec_hbm.at[pl.ds(xrow, _ROWS_LEFT)],
                        ec_v.at[pl.ds(0, _ROWS_LEFT)])
        _build_rows(_ROWS_LEFT, xrow, ee_v, ec_v, cwt_v, tbuf)
        pltpu.sync_copy(tbuf.at[pl.ds(0, _ROWS_LEFT)],
                        table_sh.at[pl.ds(xrow, _ROWS_LEFT)])

    plsc.subcore_barrier()

    gathers = [
        pltpu.async_copy(
            table_sh.at[idx_v.at[pl.ds(c * _CHUNK, _CHUNK)]],
            rows_v.at[c], gsem.at[c])
        for c in range(_N_CHUNK)
    ]
    scatters = []
    for c in range(_N_CHUNK):
        gathers[c].wait()
        scatters.append(pltpu.async_copy(
            rows_v.at[c], out_hbm.at[pl.ds(base + c * _CHUNK, _CHUNK)],
            ssem.at[c]))
    for s in scatters:
        s.wait()


def kernel(Z, element_embedding, config_weight, electron_config):
    return _sc_lookup(Z.astype(jnp.int32), element_embedding,
                      electron_config, config_weight.reshape(-1))


# speedup vs baseline: 1.3764x; 1.3764x over previous
import functools

import jax
import jax.numpy as jnp
from jax import lax
from jax.experimental import pallas as pl
from jax.experimental.pallas import tpu as pltpu
from jax.experimental.pallas import tpu_sc as plsc

ZMAX = 100
NUM_FEATURES = 128
N_ATOMS = 16384

_NC = 2
_NS = 16
_NW = _NC * _NS
_B_PER_W = N_ATOMS // _NW
_LANES = 16

_CHUNK = 512
_N_CHUNK = _B_PER_W // _CHUNK
_N_GATHER = 2


def _table_body(ee_ref, cw_ref, ec_ref, out_ref):
    proj = lax.dot_general(
        ec_ref[...], cw_ref[...],
        dimension_numbers=(((1,), (1,)), ((), ())),
        preferred_element_type=jnp.float32,
    )
    out_ref[...] = ee_ref[...] + proj


_build_table = pl.pallas_call(
    _table_body,
    out_shape=jax.ShapeDtypeStruct((ZMAX, NUM_FEATURES), jnp.float32),
)

_sc_mesh = plsc.VectorSubcoreMesh(core_axis_name="c", subcore_axis_name="s")


@functools.partial(
    pl.kernel,
    mesh=_sc_mesh,
    out_type=jax.ShapeDtypeStruct((N_ATOMS, NUM_FEATURES), jnp.float32),
    scratch_types=[
        pltpu.VMEM_SHARED((ZMAX, NUM_FEATURES), jnp.float32),
        pltpu.VMEM((_N_CHUNK, _CHUNK, NUM_FEATURES), jnp.float32),
        pltpu.VMEM((_B_PER_W,), jnp.int32),
        pltpu.SemaphoreType.DMA((_N_CHUNK,)),
        pltpu.SemaphoreType.DMA((_N_CHUNK,)),
    ],
)
def _sc_lookup(table_hbm, idx_hbm, out_hbm, table_sh, rows_v, idx_v, ssem,
               gsem):
    sid = lax.axis_index("s")
    wid = sid * _NC + lax.axis_index("c")
    base = wid * _B_PER_W
    pltpu.sync_copy(idx_hbm.at[pl.ds(base, _B_PER_W)], idx_v)

    @pl.when(sid == 0)
    def _():
        pltpu.sync_copy(table_hbm, table_sh)

    plsc.subcore_barrier()

    gathers = [
        pltpu.async_copy(
            table_sh.at[idx_v.at[pl.ds(c * _CHUNK, _CHUNK)]],
            rows_v.at[c], gsem.at[c])
        for c in range(_N_CHUNK)
    ]
    scatters = []
    for c in range(_N_CHUNK):
        gathers[c].wait()
        scatters.append(pltpu.async_copy(
            rows_v.at[c], out_hbm.at[pl.ds(base + c * _CHUNK, _CHUNK)],
            ssem.at[c]))
    for s in scatters:
        s.wait()


def kernel(Z, element_embedding, config_weight, electron_config):
    table = _build_table(element_embedding, config_weight, electron_config)
    return _sc_lookup(table, Z.astype(jnp.int32))


# speedup vs baseline: 1.4393x; 1.0457x over previous
import functools

import jax
import jax.numpy as jnp
from jax import lax
from jax.experimental import pallas as pl
from jax.experimental.pallas import tpu as pltpu
from jax.experimental.pallas import tpu_sc as plsc

ZMAX = 100
NUM_FEATURES = 128
N_ATOMS = 16384

_NC = 2
_NS = 16
_NW = _NC * _NS
_B_PER_W = N_ATOMS // _NW
_LANES = 16

_CHUNK = 64
_N_CHUNK = _B_PER_W // _CHUNK
_N_GATHER = 2


def _table_body(ee_ref, cw_ref, ec_ref, out_ref):
    proj = lax.dot_general(
        ec_ref[...], cw_ref[...],
        dimension_numbers=(((1,), (1,)), ((), ())),
        preferred_element_type=jnp.float32,
    )
    out_ref[...] = ee_ref[...] + proj


_build_table = pl.pallas_call(
    _table_body,
    out_shape=jax.ShapeDtypeStruct((ZMAX, NUM_FEATURES), jnp.float32),
)

_sc_mesh = plsc.VectorSubcoreMesh(core_axis_name="c", subcore_axis_name="s")


@functools.partial(
    pl.kernel,
    mesh=_sc_mesh,
    out_type=jax.ShapeDtypeStruct((N_ATOMS, NUM_FEATURES), jnp.float32),
    scratch_types=[
        pltpu.VMEM_SHARED((ZMAX, NUM_FEATURES), jnp.float32),
        pltpu.VMEM((_N_CHUNK, _CHUNK, NUM_FEATURES), jnp.float32),
        pltpu.VMEM((_B_PER_W,), jnp.int32),
        pltpu.SemaphoreType.DMA((_N_CHUNK,)),
        pltpu.SemaphoreType.DMA((_N_CHUNK,)),
    ],
)
def _sc_lookup(table_hbm, idx_hbm, out_hbm, table_sh, rows_v, idx_v, ssem,
               gsem):
    sid = lax.axis_index("s")
    wid = sid * _NC + lax.axis_index("c")
    base = wid * _B_PER_W
    pltpu.sync_copy(idx_hbm.at[pl.ds(base, _B_PER_W)], idx_v)

    @pl.when(sid == 0)
    def _():
        pltpu.sync_copy(table_hbm, table_sh)

    plsc.subcore_barrier()

    gathers = [
        pltpu.async_copy(
            table_sh.at[idx_v.at[pl.ds(c * _CHUNK, _CHUNK)]],
            rows_v.at[c], gsem.at[c])
        for c in range(_N_CHUNK)
    ]
    scatters = []
    for c in range(_N_CHUNK):
        gathers[c].wait()
        scatters.append(pltpu.async_copy(
            rows_v.at[c], out_hbm.at[pl.ds(base + c * _CHUNK, _CHUNK)],
            ssem.at[c]))
    for s in scatters:
        s.wait()


def kernel(Z, element_embedding, config_weight, electron_config):
    table = _build_table(element_embedding, config_weight, electron_config)
    return _sc_lookup(table, Z.astype(jnp.int32))
